# Initial kernel scaffold; baseline (speedup 1.0000x reference)
import jax
import jax.numpy as jnp
from jax.experimental import pallas as pl


def kernel(scores, pred_poses, feat, W1, b1, W2, b2):
    raise NotImplementedError("write your pallas kernel here")


# speedup vs baseline: 18.7276x; 18.7276x over previous
import dataclasses
import functools

import jax
import jax.numpy as jnp
from jax import lax
from jax.experimental import pallas as pl
from jax.experimental.pallas import tpu as pltpu
from jax.experimental.pallas import tpu_sc as plsc

B, L, K, C, H, W_ = 16, 1000, 17, 96, 64, 64
TOPK = 4
HIDDEN = 64
IN_DIM = K * (TOPK + 1)
N = B * L * K
NT = 4 * N
CHUNK = 64
NCHUNK = N // CHUNK
NW = 32
CHUNKS_PER_W = -(-NCHUNK // NW)
LANES = 16


def _sc_gather_combine(idx, w, table):
    mesh = plsc.VectorSubcoreMesh(core_axis_name="c", subcore_axis_name="s")
    cp = pltpu.CompilerParams()
    if "needs_layout_passes" in pltpu.CompilerParams.__dataclass_fields__:
        cp = dataclasses.replace(cp, needs_layout_passes=False)
    if "use_tc_tiling_on_sc" in pltpu.CompilerParams.__dataclass_fields__:
        cp = dataclasses.replace(cp, use_tc_tiling_on_sc=False)

    @functools.partial(
        pl.kernel,
        compiler_params=cp,
        out_type=jax.ShapeDtypeStruct((N, C), jnp.float32),
        mesh=mesh,
        scratch_types=[
            pltpu.VMEM((4 * CHUNK,), jnp.int32),
            pltpu.VMEM((4 * CHUNK,), jnp.float32),
            pltpu.VMEM((4 * CHUNK, C), jnp.float32),
            pltpu.VMEM((CHUNK, C), jnp.float32),
            pltpu.SemaphoreType.DMA,
        ],
    )
    def k(idx_hbm, w_hbm, table_hbm, sv_hbm, idx_v, w_v, rows_v, out_v, sem):
        wid = lax.axis_index("c") * 16 + lax.axis_index("s")

        @pl.loop(0, CHUNKS_PER_W)
        def _(j):
            cid = wid + j * NW

            @pl.when(cid < NCHUNK)
            def _():
                tap0 = cid * (4 * CHUNK)
                item0 = cid * CHUNK
                pltpu.sync_copy(idx_hbm.at[pl.ds(tap0, 4 * CHUNK)], idx_v)
                pltpu.sync_copy(w_hbm.at[pl.ds(tap0, 4 * CHUNK)], w_v)
                cps = []
                for g in range(4 * CHUNK // 128):
                    cps.append(pltpu.async_copy(
                        table_hbm.at[idx_v.at[pl.ds(128 * g, 128)]],
                        rows_v.at[pl.ds(128 * g, 128)], sem))
                for cp in cps:
                    cp.wait()

                @pl.loop(0, CHUNK)
                def _(i):
                    r0 = 4 * i
                    wv = [plsc.load_gather(w_v, [jnp.full((LANES,), r0 + t,
                                                          jnp.int32)])
                          for t in range(4)]
                    for c6 in range(C // LANES):
                        sl = pl.ds(LANES * c6, LANES)
                        acc = wv[0] * rows_v[r0, sl]
                        for t in range(1, 4):
                            acc = acc + wv[t] * rows_v[r0 + t, sl]
                        out_v[i, sl] = acc

                pltpu.sync_copy(out_v, sv_hbm.at[pl.ds(item0, CHUNK)])

    return k(idx, w, table)


ROWS = 64
GRID = B * L // ROWS


def _tc_head(sv3, scores2, W1, b1r, W2, b2r):
    def body(sv_ref, sc_ref, w1_ref, b1_ref, w2_ref, b2_ref, out_ref):
        v = sv_ref[...]
        iot = lax.broadcasted_iota(jnp.int32, v.shape, 2)
        tops = []
        for _ in range(TOPK):
            m = jnp.max(v, axis=-1, keepdims=True)
            tops.append(m)
            amax = jnp.argmax(v, axis=-1)[..., None]
            v = jnp.where(iot == amax, -jnp.inf, v)
        mean = (tops[0] + tops[1] + tops[2] + tops[3]) * 0.25
        stat = jnp.concatenate(tops + [mean], axis=-1)
        x85 = stat.reshape(ROWS, IN_DIM)
        h = lax.dot_general(x85, w1_ref[...], (((1,), (1,)), ((), ())),
                            preferred_element_type=jnp.float32) + b1_ref[...]
        h = jnp.maximum(h, 0.0)
        q = jnp.sum(h * w2_ref[...], axis=-1, keepdims=True) + b2_ref[0]
        out_ref[...] = sc_ref[...] + q

    return pl.pallas_call(
        body,
        grid=(GRID,),
        in_specs=[
            pl.BlockSpec((ROWS, K, C), lambda i: (i, 0, 0)),
            pl.BlockSpec((ROWS, 1), lambda i: (i, 0)),
            pl.BlockSpec((HIDDEN, IN_DIM), lambda i: (0, 0)),
            pl.BlockSpec((1, HIDDEN), lambda i: (0, 0)),
            pl.BlockSpec((1, HIDDEN), lambda i: (0, 0)),
            pl.BlockSpec(memory_space=pltpu.SMEM),
        ],
        out_specs=pl.BlockSpec((ROWS, 1), lambda i: (i, 0)),
        out_shape=jax.ShapeDtypeStruct((B * L, 1), jnp.float32),
    )(sv3, scores2, W1, b1r, W2, b2r)


def kernel(scores, pred_poses, feat, W1, b1, W2, b2):
    table = feat.transpose(0, 2, 3, 1).reshape(B * H * W_, C)
    pp = pred_poses.reshape(B, L, K, 2)
    ix = pp[..., 0] * W_ - 0.5
    iy = pp[..., 1] * H - 0.5
    ix0 = jnp.floor(ix)
    iy0 = jnp.floor(iy)
    wx1 = ix - ix0
    wy1 = iy - iy0
    boff = (jnp.arange(B, dtype=jnp.int32) * (H * W_))[:, None, None]
    idx_list, w_list = [], []
    for dy in (0, 1):
        for dx in (0, 1):
            xt = ix0 + dx
            yt = iy0 + dy
            valid = (xt >= 0) & (xt <= W_ - 1) & (yt >= 0) & (yt <= H - 1)
            xi = jnp.clip(xt, 0, W_ - 1).astype(jnp.int32)
            yi = jnp.clip(yt, 0, H - 1).astype(jnp.int32)
            wgt = (wx1 if dx else 1.0 - wx1) * (wy1 if dy else 1.0 - wy1)
            idx_list.append(boff + yi * W_ + xi)
            w_list.append(wgt * valid.astype(jnp.float32))
    idx = jnp.stack(idx_list, axis=-1).reshape(NT)
    w = jnp.stack(w_list, axis=-1).reshape(NT)

    sv = _sc_gather_combine(idx, w, table)
    out = _tc_head(sv.reshape(B * L, K, C), scores.reshape(B * L, 1),
                   W1, b1.reshape(1, HIDDEN), W2, b2)
    return out.reshape(B, L, 1)
